# Initial kernel scaffold; baseline (speedup 1.0000x reference)
import jax
import jax.numpy as jnp
from jax.experimental import pallas as pl


def kernel(x, edge_index, batch, W_emb, b_emb, Wc0, bc0, g0, be0, Wc1, bc1, g1, be1, Wc2, bc2, g2, be2, Wr1, br1, Wr2, br2):
    raise NotImplementedError("write your pallas kernel here")


# speedup vs baseline: 10.8957x; 10.8957x over previous
import functools

import jax
import jax.numpy as jnp
from jax import lax
from jax.experimental import pallas as pl
from jax.experimental.pallas import tpu as pltpu
from jax.experimental.pallas import tpu_sc as plsc

N = 10000
E = 320000
D = 128
H = 128
G = 16
OUT = 1

NC = 2
NS = 16
NW = NC * NS
EPW = E // NW
KE = 80
NBLK = EPW // KE
RPT = 632
NPAD = 10240
DCH = NPAD // NS


def _mesh():
    return plsc.VectorSubcoreMesh(
        core_axis_name="c", subcore_axis_name="s",
        num_cores=NC, num_subcores=NS)


@functools.partial(
    pl.kernel,
    out_type=jax.ShapeDtypeStruct((NC, NPAD), jnp.float32),
    mesh=_mesh(),
    scratch_types=[
        pltpu.VMEM((KE,), jnp.int32),
        pltpu.VMEM((KE,), jnp.float32),
        pltpu.VMEM((DCH,), jnp.float32),
        pltpu.VMEM_SHARED((NPAD,), jnp.float32),
    ],
)
def _sc_deg(dst_hbm, out_hbm, idx_v, ones_v, zb_v, deg_sh):
    c = lax.axis_index("c")
    s = lax.axis_index("s")
    wid = c * NS + s
    for j in range(KE // 16):
        ones_v[pl.ds(j * 16, 16)] = jnp.ones((16,), jnp.float32)
    for j in range(DCH // 16):
        zb_v[pl.ds(j * 16, 16)] = jnp.zeros((16,), jnp.float32)
    pltpu.sync_copy(zb_v, deg_sh.at[pl.ds(s * DCH, DCH)])
    plsc.subcore_barrier()
    base = wid * EPW

    @pl.loop(0, NBLK)
    def _edges(j):
        pltpu.sync_copy(dst_hbm.at[pl.ds(base + j * KE, KE)], idx_v)
        pltpu.sync_copy(ones_v, deg_sh.at[idx_v], add=True)

    plsc.subcore_barrier()
    pltpu.sync_copy(deg_sh.at[pl.ds(s * DCH, DCH)],
                    out_hbm.at[c].at[pl.ds(s * DCH, DCH)])


@functools.partial(
    pl.kernel,
    out_type=jax.ShapeDtypeStruct((NC, N, H), jnp.float32),
    mesh=_mesh(),
    scratch_types=[
        pltpu.VMEM((KE,), jnp.int32),
        pltpu.VMEM((KE,), jnp.int32),
        pltpu.VMEM((KE, H), jnp.float32),
        pltpu.VMEM_SHARED((N, H), jnp.float32),
        pltpu.SemaphoreType.DMA,
    ],
)
def _sc_scatter(m_hbm, src_hbm, dst_hbm, out_hbm,
                sidx_v, didx_v, rows_v, acc_sh, sem):
    c = lax.axis_index("c")
    s = lax.axis_index("s")
    wid = c * NS + s
    r0 = pl.multiple_of(jnp.where(s == NS - 1, N - RPT, s * RPT), 8)
    pltpu.sync_copy(m_hbm.at[pl.ds(r0, RPT)], acc_sh.at[pl.ds(r0, RPT)])
    plsc.subcore_barrier()
    base = wid * EPW

    @pl.loop(0, NBLK)
    def _edges(j):
        e0 = base + j * KE
        pltpu.sync_copy(src_hbm.at[pl.ds(e0, KE)], sidx_v)
        pltpu.sync_copy(dst_hbm.at[pl.ds(e0, KE)], didx_v)
        pltpu.async_copy(m_hbm.at[sidx_v], rows_v, sem).wait()
        pltpu.sync_copy(rows_v, acc_sh.at[didx_v], add=True)

    plsc.subcore_barrier()
    pltpu.sync_copy(acc_sh.at[pl.ds(r0, RPT)],
                    out_hbm.at[c].at[pl.ds(r0, RPT)])


def _lrelu(v):
    return jnp.where(v > 0, v, 0.01 * v)


def _tc_pre_body(d0_ref, d1_ref, x_ref, we_ref, be_ref, wc_ref,
                 m_ref, dis_ref):
    dis = lax.rsqrt(1.0 + d0_ref[...] + d1_ref[...])
    h = jnp.dot(x_ref[...], we_ref[...],
                preferred_element_type=jnp.float32, precision=lax.Precision.HIGHEST) + be_ref[...]
    m_ref[...] = dis * jnp.dot(h, wc_ref[...],
                               preferred_element_type=jnp.float32, precision=lax.Precision.HIGHEST)
    dis_ref[...] = dis


def _tc_mid_body(p0_ref, p1_ref, m_ref, dis_ref, bc_ref, g_ref, be_ref,
                 wc_ref, out_ref):
    dis = dis_ref[...]
    t = p0_ref[...] + p1_ref[...] - m_ref[...]
    u = dis * t + bc_ref[...]
    mu = jnp.mean(u, axis=0, keepdims=True)
    var = jnp.mean((u - mu) ** 2, axis=0, keepdims=True)
    v = (u - mu) * lax.rsqrt(var + 1e-5) * g_ref[...] + be_ref[...]
    hh = _lrelu(v)
    out_ref[...] = dis * jnp.dot(hh, wc_ref[...],
                                 preferred_element_type=jnp.float32, precision=lax.Precision.HIGHEST)


def _tc_final_body(p0_ref, p1_ref, m_ref, dis_ref, bc_ref, g_ref, be_ref,
                   batch_ref, wr1_ref, br1_ref, wr2_ref, br2_ref, out_ref):
    dis = dis_ref[...]
    t = p0_ref[...] + p1_ref[...] - m_ref[...]
    u = dis * t + bc_ref[...]
    mu = jnp.mean(u, axis=0, keepdims=True)
    var = jnp.mean((u - mu) ** 2, axis=0, keepdims=True)
    v = (u - mu) * lax.rsqrt(var + 1e-5) * g_ref[...] + be_ref[...]
    hh = _lrelu(v)
    gids = lax.broadcasted_iota(jnp.int32, (G, N), 0)
    onehot = (batch_ref[...] == gids).astype(jnp.float32)
    pooled = jnp.dot(onehot, hh, preferred_element_type=jnp.float32, precision=lax.Precision.HIGHEST)
    r = _lrelu(jnp.dot(pooled, wr1_ref[...],
                       preferred_element_type=jnp.float32, precision=lax.Precision.HIGHEST) + br1_ref[...])
    out_ref[...] = jnp.dot(r, wr2_ref[...],
                           preferred_element_type=jnp.float32, precision=lax.Precision.HIGHEST) + br2_ref[...]


def kernel(x, edge_index, batch, W_emb, b_emb, Wc0, bc0, g0, be0,
           Wc1, bc1, g1, be1, Wc2, bc2, g2, be2, Wr1, br1, Wr2, br2):
    f32 = jnp.float32
    src = edge_index[0]
    dst = edge_index[1]

    degp = _sc_deg(dst)
    d0 = degp[0, :N].reshape(N, 1)
    d1 = degp[1, :N].reshape(N, 1)

    m0, dis = pl.pallas_call(
        _tc_pre_body,
        out_shape=(jax.ShapeDtypeStruct((N, H), f32),
                   jax.ShapeDtypeStruct((N, 1), f32)),
    )(d0, d1, x, W_emb, b_emb.reshape(1, H), Wc0)

    tc_mid = pl.pallas_call(
        _tc_mid_body, out_shape=jax.ShapeDtypeStruct((N, H), f32))

    p = _sc_scatter(m0, src, dst)
    m1 = tc_mid(p[0], p[1], m0, dis, bc0.reshape(1, H), g0.reshape(1, H),
                be0.reshape(1, H), Wc1)
    p = _sc_scatter(m1, src, dst)
    m2 = tc_mid(p[0], p[1], m1, dis, bc1.reshape(1, H), g1.reshape(1, H),
                be1.reshape(1, H), Wc2)
    p = _sc_scatter(m2, src, dst)
    out = pl.pallas_call(
        _tc_final_body, out_shape=jax.ShapeDtypeStruct((G, OUT), f32),
    )(p[0], p[1], m2, dis, bc2.reshape(1, H), g2.reshape(1, H),
      be2.reshape(1, H), batch.reshape(1, N), Wr1, br1.reshape(1, H // 2),
      Wr2, br2.reshape(1, OUT))
    return out


# speedup vs baseline: 22.1151x; 2.0297x over previous
import functools

import jax
import jax.numpy as jnp
from jax import lax
from jax.experimental import pallas as pl
from jax.experimental.pallas import tpu as pltpu
from jax.experimental.pallas import tpu_sc as plsc

N = 10000
E = 320000
D = 128
H = 128
G = 16
OUT = 1

NC = 2
NS = 16
NW = NC * NS
EPW = E // NW
KE = 40
NBLK = EPW // KE
CHUNKS = 5
CBLK = NBLK // CHUNKS
NBUF = 5
LEAD = 3
RPT = 632
NPAD = 10240
DCH = NPAD // NS


def _mesh():
    return plsc.VectorSubcoreMesh(
        core_axis_name="c", subcore_axis_name="s",
        num_cores=NC, num_subcores=NS)


KED = 80
NBLKD = EPW // KED


@functools.partial(
    pl.kernel,
    out_type=jax.ShapeDtypeStruct((NC, NPAD), jnp.float32),
    mesh=_mesh(),
    scratch_types=[
        pltpu.VMEM((KED,), jnp.int32),
        pltpu.VMEM((KED,), jnp.float32),
        pltpu.VMEM((DCH,), jnp.float32),
        pltpu.VMEM_SHARED((NPAD,), jnp.float32),
    ],
)
def _sc_deg(dst_hbm, out_hbm, idx_v, ones_v, zb_v, deg_sh):
    c = lax.axis_index("c")
    s = lax.axis_index("s")
    wid = c * NS + s
    for j in range(KED // 16):
        ones_v[pl.ds(j * 16, 16)] = jnp.ones((16,), jnp.float32)
    for j in range(DCH // 16):
        zb_v[pl.ds(j * 16, 16)] = jnp.zeros((16,), jnp.float32)
    pltpu.sync_copy(zb_v, deg_sh.at[pl.ds(s * DCH, DCH)])
    plsc.subcore_barrier()
    base = wid * EPW

    @pl.loop(0, NBLKD)
    def _edges(j):
        pltpu.sync_copy(dst_hbm.at[pl.ds(base + j * KED, KED)], idx_v)
        pltpu.sync_copy(ones_v, deg_sh.at[idx_v], add=True)

    plsc.subcore_barrier()
    pltpu.sync_copy(deg_sh.at[pl.ds(s * DCH, DCH)],
                    out_hbm.at[c].at[pl.ds(s * DCH, DCH)])


@functools.partial(
    pl.kernel,
    out_type=jax.ShapeDtypeStruct((NC, N, H), jnp.float32),
    mesh=_mesh(),
    scratch_types=[
        pltpu.VMEM((2 * CBLK, KE), jnp.int32),
        pltpu.VMEM((NBUF, KE, H), jnp.float32),
        pltpu.VMEM_SHARED((N, H), jnp.float32),
        [pltpu.SemaphoreType.DMA] * NBUF,
        [pltpu.SemaphoreType.DMA] * NBUF,
    ],
)
def _sc_scatter(m_hbm, eidx_hbm, out_hbm,
                idx_v, rows_v, acc_sh, gsems, ssems):
    c = lax.axis_index("c")
    s = lax.axis_index("s")
    wid = c * NS + s
    r0 = pl.multiple_of(jnp.where(s == NS - 1, N - RPT, s * RPT), 8)
    pltpu.sync_copy(m_hbm.at[pl.ds(r0, RPT)], acc_sh.at[pl.ds(r0, RPT)])
    plsc.subcore_barrier()

    def start_gather(blk, b):
        pltpu.async_copy(m_hbm.at[idx_v.at[blk]], rows_v.at[b],
                         gsems[b])

    def wait_gather(b):
        pltpu.make_async_copy(m_hbm.at[idx_v.at[0]], rows_v.at[b],
                              gsems[b]).wait()

    def start_scatter(blk, b):
        pltpu.async_copy(rows_v.at[b], acc_sh.at[idx_v.at[CBLK + blk]],
                         ssems[b], add=True)

    def wait_scatter(b):
        pltpu.make_async_copy(rows_v.at[b], acc_sh.at[idx_v.at[CBLK]],
                              ssems[b]).wait()

    @pl.loop(0, CHUNKS)
    def _chunk(ch):
        pltpu.sync_copy(eidx_hbm.at[wid].at[ch], idx_v)
        for b in range(NBUF):
            start_gather(b, b)

        @pl.loop(0, CBLK, step=NBUF)
        def _slots(j):
            for b in range(NBUF):
                blk = j + b
                wait_gather(b)
                start_scatter(blk, b)
                bp = (b + LEAD) % NBUF

                @pl.when(jnp.logical_and(blk >= NBUF - LEAD,
                                         blk + LEAD < CBLK))
                def _():
                    wait_scatter(bp)
                    start_gather(blk + LEAD, bp)

        for b in range(NBUF):
            wait_scatter(b)

    plsc.subcore_barrier()
    pltpu.sync_copy(acc_sh.at[pl.ds(r0, RPT)],
                    out_hbm.at[c].at[pl.ds(r0, RPT)])


def _lrelu(v):
    return jnp.where(v > 0, v, 0.01 * v)


def _tc_pre_body(d0_ref, d1_ref, x_ref, we_ref, be_ref, wc_ref,
                 m_ref, dis_ref):
    dis = lax.rsqrt(1.0 + d0_ref[...] + d1_ref[...])
    h = jnp.dot(x_ref[...], we_ref[...],
                preferred_element_type=jnp.float32, precision=lax.Precision.HIGHEST) + be_ref[...]
    m_ref[...] = dis * jnp.dot(h, wc_ref[...],
                               preferred_element_type=jnp.float32, precision=lax.Precision.HIGHEST)
    dis_ref[...] = dis


def _tc_mid_body(p0_ref, p1_ref, m_ref, dis_ref, bc_ref, g_ref, be_ref,
                 wc_ref, out_ref):
    dis = dis_ref[...]
    t = p0_ref[...] + p1_ref[...] - m_ref[...]
    u = dis * t + bc_ref[...]
    mu = jnp.mean(u, axis=0, keepdims=True)
    var = jnp.mean((u - mu) ** 2, axis=0, keepdims=True)
    v = (u - mu) * lax.rsqrt(var + 1e-5) * g_ref[...] + be_ref[...]
    hh = _lrelu(v)
    out_ref[...] = dis * jnp.dot(hh, wc_ref[...],
                                 preferred_element_type=jnp.float32, precision=lax.Precision.HIGHEST)


def _tc_final_body(p0_ref, p1_ref, m_ref, dis_ref, bc_ref, g_ref, be_ref,
                   batch_ref, wr1_ref, br1_ref, wr2_ref, br2_ref, out_ref):
    dis = dis_ref[...]
    t = p0_ref[...] + p1_ref[...] - m_ref[...]
    u = dis * t + bc_ref[...]
    mu = jnp.mean(u, axis=0, keepdims=True)
    var = jnp.mean((u - mu) ** 2, axis=0, keepdims=True)
    v = (u - mu) * lax.rsqrt(var + 1e-5) * g_ref[...] + be_ref[...]
    hh = _lrelu(v)
    gids = lax.broadcasted_iota(jnp.int32, (G, N), 0)
    onehot = (batch_ref[...] == gids).astype(jnp.float32)
    pooled = jnp.dot(onehot, hh, preferred_element_type=jnp.float32, precision=lax.Precision.HIGHEST)
    r = _lrelu(jnp.dot(pooled, wr1_ref[...],
                       preferred_element_type=jnp.float32, precision=lax.Precision.HIGHEST) + br1_ref[...])
    out_ref[...] = jnp.dot(r, wr2_ref[...],
                           preferred_element_type=jnp.float32, precision=lax.Precision.HIGHEST) + br2_ref[...]


def kernel(x, edge_index, batch, W_emb, b_emb, Wc0, bc0, g0, be0,
           Wc1, bc1, g1, be1, Wc2, bc2, g2, be2, Wr1, br1, Wr2, br2):
    f32 = jnp.float32
    eidx = edge_index.reshape(2, NW, CHUNKS, CBLK, KE).transpose(1, 2, 0, 3, 4).reshape(NW, CHUNKS, 2 * CBLK, KE)

    degp = _sc_deg(edge_index[1])
    d0 = degp[0, :N].reshape(N, 1)
    d1 = degp[1, :N].reshape(N, 1)

    m0, dis = pl.pallas_call(
        _tc_pre_body,
        out_shape=(jax.ShapeDtypeStruct((N, H), f32),
                   jax.ShapeDtypeStruct((N, 1), f32)),
    )(d0, d1, x, W_emb, b_emb.reshape(1, H), Wc0)

    tc_mid = pl.pallas_call(
        _tc_mid_body, out_shape=jax.ShapeDtypeStruct((N, H), f32))

    p = _sc_scatter(m0, eidx)
    m1 = tc_mid(p[0], p[1], m0, dis, bc0.reshape(1, H), g0.reshape(1, H),
                be0.reshape(1, H), Wc1)
    p = _sc_scatter(m1, eidx)
    m2 = tc_mid(p[0], p[1], m1, dis, bc1.reshape(1, H), g1.reshape(1, H),
                be1.reshape(1, H), Wc2)
    p = _sc_scatter(m2, eidx)
    out = pl.pallas_call(
        _tc_final_body, out_shape=jax.ShapeDtypeStruct((G, OUT), f32),
    )(p[0], p[1], m2, dis, bc2.reshape(1, H), g2.reshape(1, H),
      be2.reshape(1, H), batch.reshape(1, N), Wr1, br1.reshape(1, H // 2),
      Wr2, br2.reshape(1, OUT))
    return out


# speedup vs baseline: 24.2522x; 1.0966x over previous
import functools

import jax
import jax.numpy as jnp
from jax import lax
from jax.experimental import pallas as pl
from jax.experimental.pallas import tpu as pltpu
from jax.experimental.pallas import tpu_sc as plsc

N = 10000
E = 320000
D = 128
H = 128
G = 16
OUT = 1

NC = 2
NS = 16
NW = NC * NS
EPW = E // NW
KE = 40
NBLK = EPW // KE
CHUNKS = 5
CBLK = NBLK // CHUNKS
NBUF = 5
LEAD = 3
RPT = 632
NPAD = 10240
DCH = NPAD // NS


def _mesh():
    return plsc.VectorSubcoreMesh(
        core_axis_name="c", subcore_axis_name="s",
        num_cores=NC, num_subcores=NS)


KED = 80
NBLKD = EPW // KED
NBD = 5


@functools.partial(
    pl.kernel,
    out_type=jax.ShapeDtypeStruct((NC, NPAD), jnp.float32),
    mesh=_mesh(),
    scratch_types=[
        pltpu.VMEM((NBD, KED), jnp.int32),
        pltpu.VMEM((KED,), jnp.float32),
        pltpu.VMEM((DCH,), jnp.float32),
        pltpu.VMEM_SHARED((NPAD,), jnp.float32),
        [pltpu.SemaphoreType.DMA] * NBD,
        [pltpu.SemaphoreType.DMA] * NBD,
    ],
)
def _sc_deg(dst_hbm, out_hbm, idx_v, ones_v, zb_v, deg_sh, lsems, ssems):
    c = lax.axis_index("c")
    s = lax.axis_index("s")
    wid = c * NS + s
    for j in range(KED // 16):
        ones_v[pl.ds(j * 16, 16)] = jnp.ones((16,), jnp.float32)
    for j in range(DCH // 16):
        zb_v[pl.ds(j * 16, 16)] = jnp.zeros((16,), jnp.float32)
    pltpu.sync_copy(zb_v, deg_sh.at[pl.ds(s * DCH, DCH)])
    plsc.subcore_barrier()
    base = wid * EPW

    def start_load(blk, b):
        pltpu.async_copy(dst_hbm.at[pl.ds(base + blk * KED, KED)],
                         idx_v.at[b], lsems[b])

    def wait_load(b):
        pltpu.make_async_copy(dst_hbm.at[pl.ds(base, KED)], idx_v.at[b],
                              lsems[b]).wait()

    def start_scatter(b):
        pltpu.async_copy(ones_v, deg_sh.at[idx_v.at[b]], ssems[b], add=True)

    def wait_scatter(b):
        pltpu.make_async_copy(ones_v, deg_sh.at[idx_v.at[0]], ssems[b]).wait()

    for b in range(NBD):
        start_load(b, b)

    @pl.loop(0, NBLKD, step=NBD)
    def _slots(j):
        for b in range(NBD):
            blk = j + b
            wait_load(b)
            start_scatter(b)
            bp = (b + LEAD) % NBD

            @pl.when(jnp.logical_and(blk >= NBD - LEAD,
                                     blk + LEAD < NBLKD))
            def _():
                wait_scatter(bp)
                start_load(blk + LEAD, bp)

    for b in range(NBD):
        wait_scatter(b)
    plsc.subcore_barrier()
    pltpu.sync_copy(deg_sh.at[pl.ds(s * DCH, DCH)],
                    out_hbm.at[c].at[pl.ds(s * DCH, DCH)])


@functools.partial(
    pl.kernel,
    out_type=(jax.ShapeDtypeStruct((N, H), jnp.float32),
              jax.ShapeDtypeStruct((N, H), jnp.float32)),
    mesh=_mesh(),
    scratch_types=[
        pltpu.VMEM((2 * CBLK, KE), jnp.int32),
        pltpu.VMEM((NBUF, KE, H), jnp.float32),
        pltpu.VMEM_SHARED((N, H), jnp.float32),
        [pltpu.SemaphoreType.DMA] * NBUF,
        [pltpu.SemaphoreType.DMA] * NBUF,
    ],
)
def _sc_scatter(m_hbm, eidx_hbm, out0_hbm, out1_hbm,
                idx_v, rows_v, acc_sh, gsems, ssems):
    c = lax.axis_index("c")
    s = lax.axis_index("s")
    wid = c * NS + s
    r0 = pl.multiple_of(jnp.where(s == NS - 1, N - RPT, s * RPT), 8)
    pltpu.sync_copy(m_hbm.at[pl.ds(r0, RPT)], acc_sh.at[pl.ds(r0, RPT)])
    plsc.subcore_barrier()

    def start_gather(blk, b):
        pltpu.async_copy(m_hbm.at[idx_v.at[blk]], rows_v.at[b],
                         gsems[b])

    def wait_gather(b):
        pltpu.make_async_copy(m_hbm.at[idx_v.at[0]], rows_v.at[b],
                              gsems[b]).wait()

    def start_scatter(blk, b):
        pltpu.async_copy(rows_v.at[b], acc_sh.at[idx_v.at[CBLK + blk]],
                         ssems[b], add=True)

    def wait_scatter(b):
        pltpu.make_async_copy(rows_v.at[b], acc_sh.at[idx_v.at[CBLK]],
                              ssems[b]).wait()

    @pl.loop(0, CHUNKS)
    def _chunk(ch):
        pltpu.sync_copy(eidx_hbm.at[wid].at[ch], idx_v)
        for b in range(NBUF):
            start_gather(b, b)

        @pl.loop(0, CBLK, step=NBUF)
        def _slots(j):
            for b in range(NBUF):
                blk = j + b
                wait_gather(b)
                start_scatter(blk, b)
                bp = (b + LEAD) % NBUF

                @pl.when(jnp.logical_and(blk >= NBUF - LEAD,
                                         blk + LEAD < CBLK))
                def _():
                    wait_scatter(bp)
                    start_gather(blk + LEAD, bp)

        for b in range(NBUF):
            wait_scatter(b)

    plsc.subcore_barrier()

    @pl.when(c == 0)
    def _():
        pltpu.sync_copy(acc_sh.at[pl.ds(r0, RPT)], out0_hbm.at[pl.ds(r0, RPT)])

    @pl.when(c == 1)
    def _():
        pltpu.sync_copy(acc_sh.at[pl.ds(r0, RPT)], out1_hbm.at[pl.ds(r0, RPT)])


def _lrelu(v):
    return jnp.where(v > 0, v, 0.01 * v)


def _tc_emb_body(x_ref, we_ref, be_ref, wc_ref, hw_ref):
    h = jnp.dot(x_ref[...], we_ref[...],
                preferred_element_type=jnp.float32,
                precision=lax.Precision.HIGHEST) + be_ref[...]
    hw_ref[...] = jnp.dot(h, wc_ref[...],
                          preferred_element_type=jnp.float32,
                          precision=lax.Precision.HIGHEST)


def _tc_scale_body(d0_ref, d1_ref, hw_ref, m_ref, dis_ref):
    dis = lax.rsqrt(1.0 + d0_ref[...] + d1_ref[...])
    m_ref[...] = dis * hw_ref[...]
    dis_ref[...] = dis


def _tc_mid_body(p0_ref, p1_ref, m_ref, dis_ref, bc_ref, g_ref, be_ref,
                 wc_ref, out_ref):
    dis = dis_ref[...]
    t = p0_ref[...] + p1_ref[...] - m_ref[...]
    u = dis * t + bc_ref[...]
    mu = jnp.mean(u, axis=0, keepdims=True)
    var = jnp.mean((u - mu) ** 2, axis=0, keepdims=True)
    v = (u - mu) * lax.rsqrt(var + 1e-5) * g_ref[...] + be_ref[...]
    hh = _lrelu(v)
    out_ref[...] = dis * jnp.dot(hh, wc_ref[...],
                                 preferred_element_type=jnp.float32, precision=lax.Precision.HIGHEST)


def _tc_final_body(p0_ref, p1_ref, m_ref, dis_ref, bc_ref, g_ref, be_ref,
                   batch_ref, wr1_ref, br1_ref, wr2_ref, br2_ref, out_ref):
    dis = dis_ref[...]
    t = p0_ref[...] + p1_ref[...] - m_ref[...]
    u = dis * t + bc_ref[...]
    mu = jnp.mean(u, axis=0, keepdims=True)
    var = jnp.mean((u - mu) ** 2, axis=0, keepdims=True)
    v = (u - mu) * lax.rsqrt(var + 1e-5) * g_ref[...] + be_ref[...]
    hh = _lrelu(v)
    gids = lax.broadcasted_iota(jnp.int32, (G, N), 0)
    onehot = (batch_ref[...] == gids).astype(jnp.float32)
    pooled = jnp.dot(onehot, hh, preferred_element_type=jnp.float32, precision=lax.Precision.HIGHEST)
    r = _lrelu(jnp.dot(pooled, wr1_ref[...],
                       preferred_element_type=jnp.float32, precision=lax.Precision.HIGHEST) + br1_ref[...])
    out_ref[...] = jnp.dot(r, wr2_ref[...],
                           preferred_element_type=jnp.float32, precision=lax.Precision.HIGHEST) + br2_ref[...]


def kernel(x, edge_index, batch, W_emb, b_emb, Wc0, bc0, g0, be0,
           Wc1, bc1, g1, be1, Wc2, bc2, g2, be2, Wr1, br1, Wr2, br2):
    f32 = jnp.float32
    eidx = edge_index.reshape(2, NW, CHUNKS, CBLK, KE).transpose(1, 2, 0, 3, 4).reshape(NW, CHUNKS, 2 * CBLK, KE)

    degp = _sc_deg(edge_index[1])
    d0 = degp[0, :N].reshape(N, 1)
    d1 = degp[1, :N].reshape(N, 1)

    hw0 = pl.pallas_call(
        _tc_emb_body, out_shape=jax.ShapeDtypeStruct((N, H), f32),
    )(x, W_emb, b_emb.reshape(1, H), Wc0)
    m0, dis = pl.pallas_call(
        _tc_scale_body,
        out_shape=(jax.ShapeDtypeStruct((N, H), f32),
                   jax.ShapeDtypeStruct((N, 1), f32)),
    )(d0, d1, hw0)

    tc_mid = pl.pallas_call(
        _tc_mid_body, out_shape=jax.ShapeDtypeStruct((N, H), f32))

    p0, p1 = _sc_scatter(m0, eidx)
    m1 = tc_mid(p0, p1, m0, dis, bc0.reshape(1, H), g0.reshape(1, H),
                be0.reshape(1, H), Wc1)
    p0, p1 = _sc_scatter(m1, eidx)
    m2 = tc_mid(p0, p1, m1, dis, bc1.reshape(1, H), g1.reshape(1, H),
                be1.reshape(1, H), Wc2)
    p0, p1 = _sc_scatter(m2, eidx)
    out = pl.pallas_call(
        _tc_final_body, out_shape=jax.ShapeDtypeStruct((G, OUT), f32),
    )(p0, p1, m2, dis, bc2.reshape(1, H), g2.reshape(1, H),
      be2.reshape(1, H), batch.reshape(1, N), Wr1, br1.reshape(1, H // 2),
      Wr2, br2.reshape(1, OUT))
    return out


# speedup vs baseline: 26.3038x; 1.0846x over previous
import functools

import jax
import jax.numpy as jnp
from jax import lax
from jax.experimental import pallas as pl
from jax.experimental.pallas import tpu as pltpu
from jax.experimental.pallas import tpu_sc as plsc

N = 10000
E = 320000
D = 128
H = 128
G = 16
OUT = 1

NC = 2
NS = 16
NW = NC * NS
EPW = E // NW
KE = 80
NBLK = EPW // KE
CHUNKS = 5
CBLK = NBLK // CHUNKS
NBUF = 3
LEAD = 2
RPT = 632
NPAD = 10240
DCH = NPAD // NS


def _mesh():
    return plsc.VectorSubcoreMesh(
        core_axis_name="c", subcore_axis_name="s",
        num_cores=NC, num_subcores=NS)


KED = 80
NBLKD = EPW // KED
NBD = 5
LEADD = 3


@functools.partial(
    pl.kernel,
    out_type=jax.ShapeDtypeStruct((NC, NPAD), jnp.float32),
    mesh=_mesh(),
    scratch_types=[
        pltpu.VMEM((NBD, KED), jnp.int32),
        pltpu.VMEM((KED,), jnp.float32),
        pltpu.VMEM((DCH,), jnp.float32),
        pltpu.VMEM_SHARED((NPAD,), jnp.float32),
        [pltpu.SemaphoreType.DMA] * NBD,
        [pltpu.SemaphoreType.DMA] * NBD,
    ],
)
def _sc_deg(dst_hbm, out_hbm, idx_v, ones_v, zb_v, deg_sh, lsems, ssems):
    c = lax.axis_index("c")
    s = lax.axis_index("s")
    wid = c * NS + s
    for j in range(KED // 16):
        ones_v[pl.ds(j * 16, 16)] = jnp.ones((16,), jnp.float32)
    for j in range(DCH // 16):
        zb_v[pl.ds(j * 16, 16)] = jnp.zeros((16,), jnp.float32)
    pltpu.sync_copy(zb_v, deg_sh.at[pl.ds(s * DCH, DCH)])
    plsc.subcore_barrier()
    base = wid * EPW

    def start_load(blk, b):
        pltpu.async_copy(dst_hbm.at[pl.ds(base + blk * KED, KED)],
                         idx_v.at[b], lsems[b])

    def wait_load(b):
        pltpu.make_async_copy(dst_hbm.at[pl.ds(base, KED)], idx_v.at[b],
                              lsems[b]).wait()

    def start_scatter(b):
        pltpu.async_copy(ones_v, deg_sh.at[idx_v.at[b]], ssems[b], add=True)

    def wait_scatter(b):
        pltpu.make_async_copy(ones_v, deg_sh.at[idx_v.at[0]], ssems[b]).wait()

    for b in range(NBD):
        start_load(b, b)

    @pl.loop(0, NBLKD, step=NBD)
    def _slots(j):
        for b in range(NBD):
            blk = j + b
            wait_load(b)
            start_scatter(b)
            bp = (b + LEADD) % NBD

            @pl.when(jnp.logical_and(blk >= NBD - LEADD,
                                     blk + LEADD < NBLKD))
            def _():
                wait_scatter(bp)
                start_load(blk + LEADD, bp)

    for b in range(NBD):
        wait_scatter(b)
    plsc.subcore_barrier()
    pltpu.sync_copy(deg_sh.at[pl.ds(s * DCH, DCH)],
                    out_hbm.at[c].at[pl.ds(s * DCH, DCH)])


@functools.partial(
    pl.kernel,
    out_type=(jax.ShapeDtypeStruct((N, H), jnp.float32),
              jax.ShapeDtypeStruct((N, H), jnp.float32)),
    mesh=_mesh(),
    scratch_types=[
        pltpu.VMEM((2 * CBLK, KE), jnp.int32),
        pltpu.VMEM((NBUF, KE, H), jnp.float32),
        pltpu.VMEM_SHARED((N, H), jnp.float32),
        [pltpu.SemaphoreType.DMA] * NBUF,
        [pltpu.SemaphoreType.DMA] * NBUF,
    ],
)
def _sc_scatter(m_hbm, eidx_hbm, out0_hbm, out1_hbm,
                idx_v, rows_v, acc_sh, gsems, ssems):
    c = lax.axis_index("c")
    s = lax.axis_index("s")
    wid = c * NS + s
    r0 = pl.multiple_of(jnp.where(s == NS - 1, N - RPT, s * RPT), 8)
    pltpu.sync_copy(m_hbm.at[pl.ds(r0, RPT)], acc_sh.at[pl.ds(r0, RPT)])
    plsc.subcore_barrier()

    def start_gather(blk, b):
        pltpu.async_copy(m_hbm.at[idx_v.at[blk]], rows_v.at[b],
                         gsems[b])

    def wait_gather(b):
        pltpu.make_async_copy(m_hbm.at[idx_v.at[0]], rows_v.at[b],
                              gsems[b]).wait()

    def start_scatter(blk, b):
        pltpu.async_copy(rows_v.at[b], acc_sh.at[idx_v.at[CBLK + blk]],
                         ssems[b], add=True)

    def wait_scatter(b):
        pltpu.make_async_copy(rows_v.at[b], acc_sh.at[idx_v.at[CBLK]],
                              ssems[b]).wait()

    @pl.loop(0, CHUNKS)
    def _chunk(ch):
        pltpu.sync_copy(eidx_hbm.at[wid].at[ch], idx_v)
        for b in range(NBUF):
            start_gather(b, b)

        @pl.loop(0, CBLK, step=NBUF)
        def _slots(j):
            for b in range(NBUF):
                blk = j + b

                @pl.when(blk < CBLK)
                def _():
                    wait_gather(b)
                    start_scatter(blk, b)

                bp = (b + LEAD) % NBUF

                @pl.when(jnp.logical_and(blk >= NBUF - LEAD,
                                         blk + LEAD < CBLK))
                def _():
                    wait_scatter(bp)
                    start_gather(blk + LEAD, bp)

        for b in range(NBUF):
            wait_scatter(b)

    plsc.subcore_barrier()

    @pl.when(c == 0)
    def _():
        pltpu.sync_copy(acc_sh.at[pl.ds(r0, RPT)], out0_hbm.at[pl.ds(r0, RPT)])

    @pl.when(c == 1)
    def _():
        pltpu.sync_copy(acc_sh.at[pl.ds(r0, RPT)], out1_hbm.at[pl.ds(r0, RPT)])


def _lrelu(v):
    return jnp.where(v > 0, v, 0.01 * v)


def _tc_emb_body(x_ref, we_ref, be_ref, wc_ref, hw_ref):
    h = jnp.dot(x_ref[...], we_ref[...],
                preferred_element_type=jnp.float32,
                precision=lax.Precision.HIGHEST) + be_ref[...]
    hw_ref[...] = jnp.dot(h, wc_ref[...],
                          preferred_element_type=jnp.float32,
                          precision=lax.Precision.HIGHEST)


def _tc_scale_body(d0_ref, d1_ref, hw_ref, m_ref, dis_ref):
    dis = lax.rsqrt(1.0 + d0_ref[...] + d1_ref[...])
    m_ref[...] = dis * hw_ref[...]
    dis_ref[...] = dis


def _tc_mid_body(p0_ref, p1_ref, m_ref, dis_ref, bc_ref, g_ref, be_ref,
                 wc_ref, out_ref):
    dis = dis_ref[...]
    t = p0_ref[...] + p1_ref[...] - m_ref[...]
    u = dis * t + bc_ref[...]
    mu = jnp.mean(u, axis=0, keepdims=True)
    var = jnp.mean((u - mu) ** 2, axis=0, keepdims=True)
    v = (u - mu) * lax.rsqrt(var + 1e-5) * g_ref[...] + be_ref[...]
    hh = _lrelu(v)
    out_ref[...] = dis * jnp.dot(hh, wc_ref[...],
                                 preferred_element_type=jnp.float32, precision=lax.Precision.HIGHEST)


def _tc_final_body(p0_ref, p1_ref, m_ref, dis_ref, bc_ref, g_ref, be_ref,
                   batch_ref, wr1_ref, br1_ref, wr2_ref, br2_ref, out_ref):
    dis = dis_ref[...]
    t = p0_ref[...] + p1_ref[...] - m_ref[...]
    u = dis * t + bc_ref[...]
    mu = jnp.mean(u, axis=0, keepdims=True)
    var = jnp.mean((u - mu) ** 2, axis=0, keepdims=True)
    v = (u - mu) * lax.rsqrt(var + 1e-5) * g_ref[...] + be_ref[...]
    hh = _lrelu(v)
    gids = lax.broadcasted_iota(jnp.int32, (G, N), 0)
    onehot = (batch_ref[...] == gids).astype(jnp.float32)
    pooled = jnp.dot(onehot, hh, preferred_element_type=jnp.float32, precision=lax.Precision.HIGHEST)
    r = _lrelu(jnp.dot(pooled, wr1_ref[...],
                       preferred_element_type=jnp.float32, precision=lax.Precision.HIGHEST) + br1_ref[...])
    out_ref[...] = jnp.dot(r, wr2_ref[...],
                           preferred_element_type=jnp.float32, precision=lax.Precision.HIGHEST) + br2_ref[...]


def kernel(x, edge_index, batch, W_emb, b_emb, Wc0, bc0, g0, be0,
           Wc1, bc1, g1, be1, Wc2, bc2, g2, be2, Wr1, br1, Wr2, br2):
    f32 = jnp.float32
    eidx = edge_index.reshape(2, NW, CHUNKS, CBLK, KE).transpose(1, 2, 0, 3, 4).reshape(NW, CHUNKS, 2 * CBLK, KE)

    degp = _sc_deg(edge_index[1])
    d0 = degp[0, :N].reshape(N, 1)
    d1 = degp[1, :N].reshape(N, 1)

    hw0 = pl.pallas_call(
        _tc_emb_body, out_shape=jax.ShapeDtypeStruct((N, H), f32),
    )(x, W_emb, b_emb.reshape(1, H), Wc0)
    m0, dis = pl.pallas_call(
        _tc_scale_body,
        out_shape=(jax.ShapeDtypeStruct((N, H), f32),
                   jax.ShapeDtypeStruct((N, 1), f32)),
    )(d0, d1, hw0)

    tc_mid = pl.pallas_call(
        _tc_mid_body, out_shape=jax.ShapeDtypeStruct((N, H), f32))

    p0, p1 = _sc_scatter(m0, eidx)
    m1 = tc_mid(p0, p1, m0, dis, bc0.reshape(1, H), g0.reshape(1, H),
                be0.reshape(1, H), Wc1)
    p0, p1 = _sc_scatter(m1, eidx)
    m2 = tc_mid(p0, p1, m1, dis, bc1.reshape(1, H), g1.reshape(1, H),
                be1.reshape(1, H), Wc2)
    p0, p1 = _sc_scatter(m2, eidx)
    out = pl.pallas_call(
        _tc_final_body, out_shape=jax.ShapeDtypeStruct((G, OUT), f32),
    )(p0, p1, m2, dis, bc2.reshape(1, H), g2.reshape(1, H),
      be2.reshape(1, H), batch.reshape(1, N), Wr1, br1.reshape(1, H // 2),
      Wr2, br2.reshape(1, OUT))
    return out
